# Initial kernel scaffold; baseline (speedup 1.0000x reference)
import jax
import jax.numpy as jnp
from jax.experimental import pallas as pl


def kernel(tokens, table):
    raise NotImplementedError("write your pallas kernel here")


# speedup vs baseline: 2.4153x; 2.4153x over previous
import functools
import math

import jax
import jax.numpy as jnp
from jax import lax
from jax.experimental import pallas as pl
from jax.experimental.pallas import tpu as pltpu
from jax.experimental.pallas import tpu_sc as plsc

VOCAB_SIZE = 100000
D = 128
SCALE = math.sqrt(D)

NC = 2
NS = 16
NW = NC * NS
LANES = 16

CHUNK = 128
B_TOTAL = 4096 * 50
B_PER_W = B_TOTAL // NW
N_CHUNKS = B_PER_W // CHUNK


def _body(tok_hbm, table_hbm, out_hbm, idx_v, rows_v, sem):
    wid = lax.axis_index("s") * NC + lax.axis_index("c")
    base = wid * B_PER_W

    pltpu.sync_copy(tok_hbm.at[wid], idx_v)

    @pl.loop(0, N_CHUNKS)
    def _chunk(j):
        pltpu.async_copy(table_hbm.at[idx_v.at[j]], rows_v, sem).wait()

        @pl.loop(0, CHUNK)
        def _row(r):
            for k in range(D // LANES):
                sl = pl.ds(k * LANES, LANES)
                rows_v[r, sl] = rows_v[r, sl] * SCALE

        pltpu.sync_copy(rows_v, out_hbm.at[pl.ds(base + j * CHUNK, CHUNK)])


@functools.partial(jax.jit, static_argnums=())
def _embed(tokens3d, table):
    mesh = plsc.VectorSubcoreMesh(
        core_axis_name="c", subcore_axis_name="s",
        num_cores=NC, num_subcores=NS,
    )
    kern = pl.kernel(
        _body,
        out_type=jax.ShapeDtypeStruct((B_TOTAL, D), jnp.float32),
        mesh=mesh,
        scratch_types=[
            pltpu.VMEM((N_CHUNKS, CHUNK), jnp.int32),
            pltpu.VMEM((CHUNK, D), jnp.float32),
            pltpu.SemaphoreType.DMA,
        ],
    )
    return kern(tokens3d, table)


def kernel(tokens, table):
    tok = tokens.astype(jnp.int32).reshape(NW, N_CHUNKS, CHUNK)
    out = _embed(tok, table)
    return out.reshape(tokens.shape[0], tokens.shape[1], D)
